# baseline (speedup 1.0000x reference)
import functools

import jax
import jax.numpy as jnp
from jax import lax
from jax.experimental import pallas as pl
from jax.experimental.pallas import tpu as pltpu
from jax.experimental.pallas import tpu_sc as plsc

_B = 4096
_D = 128
_HALF = _B // 2
_LOG2E = 1.4426950408889634


def _sc_gather_main(user_id, product_id, user_table, product_table):
    info = plsc.get_sparse_core_info()
    nw = info.num_cores * info.num_subcores
    ppw = _B // nw
    upw = _HALF // nw
    mesh = plsc.VectorSubcoreMesh(core_axis_name="c", subcore_axis_name="s")

    @functools.partial(
        pl.kernel,
        out_type=(
            jax.ShapeDtypeStruct((_B, _D), jnp.float32),
            jax.ShapeDtypeStruct((_HALF, _D), jnp.float32),
        ),
        mesh=mesh,
        scratch_types=(
            pltpu.VMEM((ppw,), jnp.int32),
            pltpu.VMEM((ppw, _D), jnp.float32),
            pltpu.VMEM((upw,), jnp.int32),
            pltpu.VMEM((upw, _D), jnp.float32),
            pltpu.SemaphoreType.DMA,
            pltpu.SemaphoreType.DMA,
            pltpu.SemaphoreType.DMA,
            pltpu.SemaphoreType.DMA,
        ),
    )
    def gather(uid_hbm, pid_hbm, utab_hbm, ptab_hbm, pout_hbm, uout_hbm,
               pidx, prows, uidx, urows, psem, usem, s1, s2):
        wid = lax.axis_index("s") * info.num_cores + lax.axis_index("c")
        pbase = wid * ppw
        ubase = wid * upw
        ci = pltpu.async_copy(pid_hbm.at[pl.ds(pbase, ppw)], pidx, s1)
        cj = pltpu.async_copy(uid_hbm.at[pl.ds(ubase, upw)], uidx, s2)
        ci.wait()
        cp = pltpu.async_copy(ptab_hbm.at[pidx], prows, psem)
        cj.wait()
        cu = pltpu.async_copy(utab_hbm.at[uidx], urows, usem)
        cp.wait()
        sp = pltpu.async_copy(prows, pout_hbm.at[pl.ds(pbase, ppw)], s1)
        cu.wait()
        su = pltpu.async_copy(urows, uout_hbm.at[pl.ds(ubase, upw)], s2)
        sp.wait()
        su.wait()

    return gather(user_id, product_id, user_table, product_table)


def _sc_gather_tail(user_id, user_table):
    info = plsc.get_sparse_core_info()
    nw = info.num_cores * info.num_subcores
    upw = _HALF // nw
    mesh = plsc.VectorSubcoreMesh(core_axis_name="c", subcore_axis_name="s")

    @functools.partial(
        pl.kernel,
        out_type=jax.ShapeDtypeStruct((_HALF, _D), jnp.float32),
        mesh=mesh,
        scratch_types=(
            pltpu.VMEM((upw,), jnp.int32),
            pltpu.VMEM((upw, _D), jnp.float32),
            pltpu.SemaphoreType.DMA,
            pltpu.SemaphoreType.DMA,
        ),
    )
    def gather(uid_hbm, utab_hbm, uout_hbm, uidx, urows, usem, s1):
        wid = lax.axis_index("s") * info.num_cores + lax.axis_index("c")
        base = wid * upw
        pltpu.async_copy(uid_hbm.at[pl.ds(_HALF + base, upw)], uidx, s1).wait()
        pltpu.async_copy(utab_hbm.at[uidx], urows, usem).wait()
        pltpu.sync_copy(urows, uout_hbm.at[pl.ds(base, upw)])

    return gather(user_id, user_table)


def _half_loss(u_ref, pall_ref, pdiag_ref):
    l2t = lax.dot_general(
        pall_ref[...].astype(jnp.bfloat16),
        (u_ref[...] * _LOG2E).astype(jnp.bfloat16),
        (((1,), (1,)), ((), ())),
        preferred_element_type=jnp.float32,
    )
    s = jnp.sum(jnp.exp2(l2t), axis=0)
    return jnp.sum(jnp.log(s)) - jnp.sum(u_ref[...] * pdiag_ref[...])


def _loss_body0(u_ref, pall_ref, pdiag_ref, acc_ref):
    acc_ref[0, 0] = _half_loss(u_ref, pall_ref, pdiag_ref)


def _loss_body1(u_ref, pall_ref, pdiag_ref, accin_ref, acc_ref):
    acc_ref[0, 0] = accin_ref[0, 0] + _half_loss(u_ref, pall_ref, pdiag_ref)


def _tc_loss(u_half, p_emb, half_idx, acc_prev=None):
    in_specs = [
        pl.BlockSpec((_HALF, _D), lambda i: (0, 0)),
        pl.BlockSpec((_B, _D), lambda i: (0, 0)),
        pl.BlockSpec((_HALF, _D), lambda i, h=half_idx: (h, 0)),
    ]
    args = [u_half, p_emb, p_emb]
    body = _loss_body0
    if acc_prev is not None:
        in_specs.append(pl.BlockSpec(memory_space=pltpu.SMEM))
        args.append(acc_prev)
        body = _loss_body1
    return pl.pallas_call(
        body,
        grid=(1,),
        in_specs=in_specs,
        out_specs=pl.BlockSpec(memory_space=pltpu.SMEM),
        out_shape=jax.ShapeDtypeStruct((1, 1), jnp.float32),
    )(*args)


def kernel(user_id, product_id, user_table, product_table):
    p_emb, u1 = _sc_gather_main(user_id, product_id, user_table, product_table)
    u2 = _sc_gather_tail(user_id, user_table)
    acc1 = _tc_loss(u1, p_emb, 0)
    acc2 = _tc_loss(u2, p_emb, 1, acc1)
    return acc2[0, 0]


# speedup vs baseline: 1.0840x; 1.0840x over previous
import functools

import jax
import jax.numpy as jnp
from jax import lax
from jax.experimental import pallas as pl
from jax.experimental.pallas import tpu as pltpu
from jax.experimental.pallas import tpu_sc as plsc

_B = 4096
_D = 128
_BLK = 1024
_LOG2E = 1.4426950408889634


def _sc_gather(user_id, product_id, user_table, product_table):
    info = plsc.get_sparse_core_info()
    nw = info.num_cores * info.num_subcores
    bpw = _B // nw
    mesh = plsc.VectorSubcoreMesh(core_axis_name="c", subcore_axis_name="s")

    @functools.partial(
        pl.kernel,
        out_type=(
            jax.ShapeDtypeStruct((_B, _D), jnp.float32),
            jax.ShapeDtypeStruct((_B, _D), jnp.float32),
        ),
        mesh=mesh,
        scratch_types=(
            pltpu.VMEM((bpw,), jnp.int32),
            pltpu.VMEM((bpw, _D), jnp.float32),
            pltpu.VMEM((bpw,), jnp.int32),
            pltpu.VMEM((bpw, _D), jnp.float32),
            pltpu.SemaphoreType.DMA,
            pltpu.SemaphoreType.DMA,
            pltpu.SemaphoreType.DMA,
            pltpu.SemaphoreType.DMA,
        ),
    )
    def gather(uid_hbm, pid_hbm, utab_hbm, ptab_hbm, uout_hbm, pout_hbm,
               uidx, urows, pidx, prows, usem, psem, s1, s2):
        wid = lax.axis_index("s") * info.num_cores + lax.axis_index("c")
        base = wid * bpw
        ci = pltpu.async_copy(pid_hbm.at[pl.ds(base, bpw)], pidx, s1)
        cj = pltpu.async_copy(uid_hbm.at[pl.ds(base, bpw)], uidx, s2)
        ci.wait()
        cp = pltpu.async_copy(ptab_hbm.at[pidx], prows, psem)
        cj.wait()
        cu = pltpu.async_copy(utab_hbm.at[uidx], urows, usem)
        cp.wait()
        sp = pltpu.async_copy(prows, pout_hbm.at[pl.ds(base, bpw)], s1)
        cu.wait()
        su = pltpu.async_copy(urows, uout_hbm.at[pl.ds(base, bpw)], s2)
        sp.wait()
        su.wait()

    return gather(user_id, product_id, user_table, product_table)


def _loss_body(u_ref, pall_ref, acc_ref):
    i = pl.program_id(0)
    l2t = lax.dot_general(
        pall_ref[...].astype(jnp.bfloat16),
        (u_ref[...] * _LOG2E).astype(jnp.bfloat16),
        (((1,), (1,)), ((), ())),
        preferred_element_type=jnp.float32,
    )
    s = jnp.sum(jnp.exp2(l2t), axis=0)
    pdiag = pall_ref[pl.ds(i * _BLK, _BLK), :]
    part = jnp.sum(jnp.log(s)) - jnp.sum(u_ref[...] * pdiag)

    @pl.when(i == 0)
    def _init():
        acc_ref[0, 0] = jnp.float32(0.0)

    acc_ref[0, 0] += part


def _tc_loss(u_emb, p_emb):
    out = pl.pallas_call(
        _loss_body,
        grid=(_B // _BLK,),
        in_specs=[
            pl.BlockSpec((_BLK, _D), lambda i: (i, 0)),
            pl.BlockSpec((_B, _D), lambda i: (0, 0)),
        ],
        out_specs=pl.BlockSpec(memory_space=pltpu.SMEM),
        out_shape=jax.ShapeDtypeStruct((1, 1), jnp.float32),
    )(u_emb, p_emb)
    return out[0, 0]


def kernel(user_id, product_id, user_table, product_table):
    u_emb, p_emb = _sc_gather(user_id, product_id, user_table, product_table)
    return _tc_loss(u_emb, p_emb)


# speedup vs baseline: 1.1021x; 1.0167x over previous
import functools

import jax
import jax.numpy as jnp
from jax import lax
from jax.experimental import pallas as pl
from jax.experimental.pallas import tpu as pltpu
from jax.experimental.pallas import tpu_sc as plsc

_B = 4096
_D = 128
_BLK = 2048
_LOG2E = 1.4426950408889634


def _sc_gather(user_id, product_id, user_table, product_table):
    info = plsc.get_sparse_core_info()
    nw = info.num_cores * info.num_subcores
    bpw = _B // nw
    mesh = plsc.VectorSubcoreMesh(core_axis_name="c", subcore_axis_name="s")

    @functools.partial(
        pl.kernel,
        out_type=(
            jax.ShapeDtypeStruct((_B, _D), jnp.float32),
            jax.ShapeDtypeStruct((_B, _D), jnp.float32),
        ),
        mesh=mesh,
        scratch_types=(
            pltpu.VMEM((bpw,), jnp.int32),
            pltpu.VMEM((bpw, _D), jnp.float32),
            pltpu.VMEM((bpw,), jnp.int32),
            pltpu.VMEM((bpw, _D), jnp.float32),
            pltpu.SemaphoreType.DMA,
            pltpu.SemaphoreType.DMA,
            pltpu.SemaphoreType.DMA,
            pltpu.SemaphoreType.DMA,
        ),
    )
    def gather(uid_hbm, pid_hbm, utab_hbm, ptab_hbm, uout_hbm, pout_hbm,
               uidx, urows, pidx, prows, usem, psem, s1, s2):
        wid = lax.axis_index("s") * info.num_cores + lax.axis_index("c")
        base = wid * bpw
        ci = pltpu.async_copy(pid_hbm.at[pl.ds(base, bpw)], pidx, s1)
        cj = pltpu.async_copy(uid_hbm.at[pl.ds(base, bpw)], uidx, s2)
        ci.wait()
        cp = pltpu.async_copy(ptab_hbm.at[pidx], prows, psem)
        cj.wait()
        cu = pltpu.async_copy(utab_hbm.at[uidx], urows, usem)
        cp.wait()
        sp = pltpu.async_copy(prows, pout_hbm.at[pl.ds(base, bpw)], s1)
        cu.wait()
        su = pltpu.async_copy(urows, uout_hbm.at[pl.ds(base, bpw)], s2)
        sp.wait()
        su.wait()

    return gather(user_id, product_id, user_table, product_table)


def _loss_body(u_ref, pall_ref, acc_ref):
    i = pl.program_id(0)
    l2t = lax.dot_general(
        pall_ref[...].astype(jnp.bfloat16),
        (u_ref[...] * _LOG2E).astype(jnp.bfloat16),
        (((1,), (1,)), ((), ())),
        preferred_element_type=jnp.float32,
    )
    s = jnp.sum(jnp.exp2(l2t), axis=0)
    pdiag = pall_ref[pl.ds(i * _BLK, _BLK), :]
    part = jnp.sum(jnp.log(s)) - jnp.sum(u_ref[...] * pdiag)

    @pl.when(i == 0)
    def _init():
        acc_ref[0, 0] = jnp.float32(0.0)

    acc_ref[0, 0] += part


def _tc_loss(u_emb, p_emb):
    out = pl.pallas_call(
        _loss_body,
        grid=(_B // _BLK,),
        in_specs=[
            pl.BlockSpec((_BLK, _D), lambda i: (i, 0)),
            pl.BlockSpec((_B, _D), lambda i: (0, 0)),
        ],
        out_specs=pl.BlockSpec(memory_space=pltpu.SMEM),
        out_shape=jax.ShapeDtypeStruct((1, 1), jnp.float32),
    )(u_emb, p_emb)
    return out[0, 0]


def kernel(user_id, product_id, user_table, product_table):
    u_emb, p_emb = _sc_gather(user_id, product_id, user_table, product_table)
    return _tc_loss(u_emb, p_emb)


# speedup vs baseline: 1.1053x; 1.0028x over previous
import functools

import jax
import jax.numpy as jnp
from jax import lax
from jax.experimental import pallas as pl
from jax.experimental.pallas import tpu as pltpu
from jax.experimental.pallas import tpu_sc as plsc

_B = 4096
_D = 128
_BLK = 2048
_LOG2E = 1.4426950408889634


def _sc_gather(user_id, product_id, user_table, product_table):
    info = plsc.get_sparse_core_info()
    nw = info.num_cores * info.num_subcores
    bpw = _B // nw
    mesh = plsc.VectorSubcoreMesh(core_axis_name="c", subcore_axis_name="s")
    half = bpw // 2

    @functools.partial(
        pl.kernel,
        out_type=(
            jax.ShapeDtypeStruct((_B, _D), jnp.float32),
            jax.ShapeDtypeStruct((_B, _D), jnp.float32),
        ),
        mesh=mesh,
        scratch_types=(
            pltpu.VMEM((bpw,), jnp.int32),
            pltpu.VMEM((bpw, _D), jnp.float32),
            pltpu.VMEM((bpw,), jnp.int32),
            pltpu.VMEM((bpw, _D), jnp.float32),
            pltpu.SemaphoreType.DMA,
            pltpu.SemaphoreType.DMA,
            pltpu.SemaphoreType.DMA,
            pltpu.SemaphoreType.DMA,
        ),
    )
    def gather(uid_hbm, pid_hbm, utab_hbm, ptab_hbm, uout_hbm, pout_hbm,
               uidx, urows, pidx, prows, usem, psem, s1, s2):
        wid = lax.axis_index("s") * info.num_cores + lax.axis_index("c")
        base = wid * bpw
        ci = pltpu.async_copy(pid_hbm.at[pl.ds(base, bpw)], pidx, s1)
        cj = pltpu.async_copy(uid_hbm.at[pl.ds(base, bpw)], uidx, s2)
        ci.wait()
        cp0 = pltpu.async_copy(ptab_hbm.at[pidx.at[pl.ds(0, half)]],
                               prows.at[pl.ds(0, half)], psem)
        cp1 = pltpu.async_copy(ptab_hbm.at[pidx.at[pl.ds(half, half)]],
                               prows.at[pl.ds(half, half)], psem)
        cj.wait()
        cu0 = pltpu.async_copy(utab_hbm.at[uidx.at[pl.ds(0, half)]],
                               urows.at[pl.ds(0, half)], usem)
        cu1 = pltpu.async_copy(utab_hbm.at[uidx.at[pl.ds(half, half)]],
                               urows.at[pl.ds(half, half)], usem)
        cp0.wait()
        sp0 = pltpu.async_copy(prows.at[pl.ds(0, half)],
                               pout_hbm.at[pl.ds(base, half)], s1)
        cp1.wait()
        sp1 = pltpu.async_copy(prows.at[pl.ds(half, half)],
                               pout_hbm.at[pl.ds(base + half, half)], s2)
        cu0.wait()
        su0 = pltpu.async_copy(urows.at[pl.ds(0, half)],
                               uout_hbm.at[pl.ds(base, half)], s1)
        cu1.wait()
        su1 = pltpu.async_copy(urows.at[pl.ds(half, half)],
                               uout_hbm.at[pl.ds(base + half, half)], s2)
        sp0.wait()
        sp1.wait()
        su0.wait()
        su1.wait()

    return gather(user_id, product_id, user_table, product_table)


def _loss_body(u_ref, pall_ref, acc_ref):
    i = pl.program_id(0)
    l2t = lax.dot_general(
        pall_ref[...].astype(jnp.bfloat16),
        (u_ref[...] * _LOG2E).astype(jnp.bfloat16),
        (((1,), (1,)), ((), ())),
        preferred_element_type=jnp.float32,
    )
    s = jnp.sum(jnp.exp2(l2t), axis=0)
    pdiag = pall_ref[pl.ds(i * _BLK, _BLK), :]
    part = jnp.sum(jnp.log(s)) - jnp.sum(u_ref[...] * pdiag)

    @pl.when(i == 0)
    def _init():
        acc_ref[0, 0] = jnp.float32(0.0)

    acc_ref[0, 0] += part


def _tc_loss(u_emb, p_emb):
    out = pl.pallas_call(
        _loss_body,
        grid=(_B // _BLK,),
        in_specs=[
            pl.BlockSpec((_BLK, _D), lambda i: (i, 0)),
            pl.BlockSpec((_B, _D), lambda i: (0, 0)),
        ],
        out_specs=pl.BlockSpec(memory_space=pltpu.SMEM),
        out_shape=jax.ShapeDtypeStruct((1, 1), jnp.float32),
    )(u_emb, p_emb)
    return out[0, 0]


def kernel(user_id, product_id, user_table, product_table):
    u_emb, p_emb = _sc_gather(user_id, product_id, user_table, product_table)
    return _tc_loss(u_emb, p_emb)
